# Initial kernel scaffold; baseline (speedup 1.0000x reference)
import jax
import jax.numpy as jnp
from jax.experimental import pallas as pl


def kernel(x, table):
    raise NotImplementedError("write your pallas kernel here")


# speedup vs baseline: 1.8315x; 1.8315x over previous
import functools

import jax
import jax.numpy as jnp
from jax import lax
from jax.experimental import pallas as pl
from jax.experimental.pallas import tpu as pltpu
from jax.experimental.pallas import tpu_sc as plsc

BATCH = 16384
FIELDS = 26
EMBED = 32
N_IDX = BATCH * FIELDS
LANES = 16

CHUNK = 128
NBUF = 4


def _make_sc_kernel(nw, per_w):
    n_chunks = per_w // CHUNK
    mesh = plsc.VectorSubcoreMesh(core_axis_name="c", subcore_axis_name="s")
    nc = mesh.num_cores

    @functools.partial(
        pl.kernel,
        out_type=jax.ShapeDtypeStruct((nw, LANES), jnp.float32),
        mesh=mesh,
        compiler_params=pltpu.CompilerParams(use_tc_tiling_on_sc=False),
        scratch_types=[
            pltpu.VMEM((n_chunks, CHUNK), jnp.int32),
            pltpu.VMEM((CHUNK, EMBED), jnp.float32),
            pltpu.VMEM((CHUNK, EMBED), jnp.float32),
            pltpu.VMEM((CHUNK, EMBED), jnp.float32),
            pltpu.VMEM((CHUNK, EMBED), jnp.float32),
            pltpu.VMEM((LANES,), jnp.float32),
            pltpu.SemaphoreType.DMA,
            pltpu.SemaphoreType.DMA,
            pltpu.SemaphoreType.DMA,
            pltpu.SemaphoreType.DMA,
        ],
    )
    def sc_kernel(idx_hbm, table_hbm, out_hbm,
                  idx_v, buf0, buf1, buf2, buf3, outv,
                  sem0, sem1, sem2, sem3):
        bufs = (buf0, buf1, buf2, buf3)
        sems = (sem0, sem1, sem2, sem3)
        wid = lax.axis_index("s") * nc + lax.axis_index("c")

        pltpu.sync_copy(idx_hbm.at[wid], idx_v)
        for b in range(NBUF):
            pltpu.async_copy(table_hbm.at[idx_v.at[b]], bufs[b], sems[b])

        zero = jnp.zeros((LANES,), jnp.float32)

        def group(g, carry):
            acc0, acc1 = carry
            for b in range(NBUF):
                j = g * NBUF + b
                pltpu.make_async_copy(
                    table_hbm.at[idx_v.at[j]], bufs[b], sems[b]).wait()
                c0 = bufs[b][0, 0:LANES]
                c1 = bufs[b][0, LANES:EMBED]
                for r in range(1, CHUNK):
                    c0 = c0 + bufs[b][r, 0:LANES]
                    c1 = c1 + bufs[b][r, LANES:EMBED]
                acc0 = acc0 + c0
                acc1 = acc1 + c1
                nj = j + NBUF

                @pl.when(nj < n_chunks)
                def _():
                    pltpu.async_copy(
                        table_hbm.at[idx_v.at[nj]], bufs[b], sems[b])
            return acc0, acc1

        acc0, acc1 = lax.fori_loop(0, n_chunks // NBUF, group, (zero, zero))
        outv[...] = acc0 + acc1
        pltpu.sync_copy(outv, out_hbm.at[wid])

    return sc_kernel


def kernel(x, table):
    nw = 32
    per_w = N_IDX // nw
    idx = x.astype(jnp.int32).reshape(nw, per_w // CHUNK, CHUNK)
    partials = _make_sc_kernel(nw, per_w)(idx, table)
    return jnp.sum(partials) / jnp.float32(N_IDX * EMBED)


# speedup vs baseline: 10.3943x; 5.6752x over previous
import functools

import jax
import jax.numpy as jnp
from jax import lax
from jax.experimental import pallas as pl
from jax.experimental.pallas import tpu as pltpu
from jax.experimental.pallas import tpu_sc as plsc

BATCH = 16384
FIELDS = 26
EMBED = 32
VOCAB = 1000000
N_IDX = BATCH * FIELDS
LANES = 16

CHUNK = 128
NBUF = 4
TC_BN = 32768


def _rowsum_tc(table_t):
    def body(t_ref, o_ref):
        o_ref[...] = jnp.sum(t_ref[...], axis=0)

    return pl.pallas_call(
        body,
        grid=(pl.cdiv(VOCAB, TC_BN),),
        in_specs=[pl.BlockSpec((EMBED, TC_BN), lambda i: (0, i))],
        out_specs=pl.BlockSpec((TC_BN,), lambda i: (i,)),
        out_shape=jax.ShapeDtypeStruct((VOCAB,), jnp.float32),
    )(table_t)


def _make_sc_kernel(nw, per_w):
    n_chunks = per_w // CHUNK
    mesh = plsc.VectorSubcoreMesh(core_axis_name="c", subcore_axis_name="s")
    nc = mesh.num_cores

    @functools.partial(
        pl.kernel,
        out_type=jax.ShapeDtypeStruct((nw, LANES), jnp.float32),
        mesh=mesh,
        compiler_params=pltpu.CompilerParams(use_tc_tiling_on_sc=False),
        scratch_types=[
            pltpu.VMEM((n_chunks, CHUNK), jnp.int32),
            pltpu.VMEM((CHUNK,), jnp.float32),
            pltpu.VMEM((CHUNK,), jnp.float32),
            pltpu.VMEM((CHUNK,), jnp.float32),
            pltpu.VMEM((CHUNK,), jnp.float32),
            pltpu.VMEM((LANES,), jnp.float32),
            pltpu.SemaphoreType.DMA,
            pltpu.SemaphoreType.DMA,
            pltpu.SemaphoreType.DMA,
            pltpu.SemaphoreType.DMA,
        ],
    )
    def sc_kernel(idx_hbm, rowsum_hbm, out_hbm,
                  idx_v, buf0, buf1, buf2, buf3, outv,
                  sem0, sem1, sem2, sem3):
        bufs = (buf0, buf1, buf2, buf3)
        sems = (sem0, sem1, sem2, sem3)
        wid = lax.axis_index("s") * nc + lax.axis_index("c")

        pltpu.sync_copy(idx_hbm.at[wid], idx_v)
        for b in range(NBUF):
            pltpu.async_copy(rowsum_hbm.at[idx_v.at[b]], bufs[b], sems[b])

        zero = jnp.zeros((LANES,), jnp.float32)

        def group(g, acc):
            for b in range(NBUF):
                j = g * NBUF + b
                pltpu.make_async_copy(
                    rowsum_hbm.at[idx_v.at[j]], bufs[b], sems[b]).wait()
                c = bufs[b][0:LANES]
                for r in range(1, CHUNK // LANES):
                    c = c + bufs[b][r * LANES:(r + 1) * LANES]
                acc = acc + c
                nj = j + NBUF

                @pl.when(nj < n_chunks)
                def _():
                    pltpu.async_copy(
                        rowsum_hbm.at[idx_v.at[nj]], bufs[b], sems[b])
            return acc

        acc = lax.fori_loop(0, n_chunks // NBUF, group, zero)
        outv[...] = acc
        pltpu.sync_copy(outv, out_hbm.at[wid])

    return sc_kernel


def kernel(x, table):
    nw = 32
    per_w = N_IDX // nw
    rowsum = _rowsum_tc(table.T)
    idx = x.astype(jnp.int32).reshape(nw, per_w // CHUNK, CHUNK)
    partials = _make_sc_kernel(nw, per_w)(idx, rowsum)
    return jnp.sum(partials) / jnp.float32(N_IDX * EMBED)


# speedup vs baseline: 12.2398x; 1.1776x over previous
import functools

import jax
import jax.numpy as jnp
from jax import lax
from jax.experimental import pallas as pl
from jax.experimental.pallas import tpu as pltpu
from jax.experimental.pallas import tpu_sc as plsc

BATCH = 16384
FIELDS = 26
EMBED = 32
VOCAB = 1000000
N_IDX = BATCH * FIELDS
LANES = 16

CHUNK = 128
NBUF = 4
TC_BN = 32768


def _rowsum_tc(table_t):
    def body(t_ref, o_ref):
        o_ref[...] = jnp.sum(t_ref[...], axis=0)

    return pl.pallas_call(
        body,
        grid=(pl.cdiv(VOCAB, TC_BN),),
        in_specs=[pl.BlockSpec((EMBED, TC_BN), lambda i: (0, i))],
        out_specs=pl.BlockSpec((TC_BN,), lambda i: (i,)),
        out_shape=jax.ShapeDtypeStruct((VOCAB,), jnp.float32),
    )(table_t)


def _make_sc_kernel(nw):
    cols_w = BATCH // nw
    n_chunks = FIELDS * (cols_w // CHUNK)
    k_per_row = cols_w // CHUNK
    mesh = plsc.VectorSubcoreMesh(core_axis_name="c", subcore_axis_name="s")
    nc = mesh.num_cores

    @functools.partial(
        pl.kernel,
        out_type=jax.ShapeDtypeStruct((nw, LANES), jnp.float32),
        mesh=mesh,
        compiler_params=pltpu.CompilerParams(use_tc_tiling_on_sc=True),
        scratch_types=[
            pltpu.VMEM((FIELDS, cols_w), jnp.int32),
            pltpu.VMEM((CHUNK,), jnp.float32),
            pltpu.VMEM((CHUNK,), jnp.float32),
            pltpu.VMEM((CHUNK,), jnp.float32),
            pltpu.VMEM((CHUNK,), jnp.float32),
            pltpu.VMEM((LANES,), jnp.float32),
            pltpu.SemaphoreType.DMA,
            pltpu.SemaphoreType.DMA,
            pltpu.SemaphoreType.DMA,
            pltpu.SemaphoreType.DMA,
        ],
    )
    def sc_kernel(idx_hbm, rowsum_hbm, out_hbm,
                  idx_v, buf0, buf1, buf2, buf3, outv,
                  sem0, sem1, sem2, sem3):
        bufs = (buf0, buf1, buf2, buf3)
        sems = (sem0, sem1, sem2, sem3)
        wid = lax.axis_index("s") * nc + lax.axis_index("c")

        pltpu.sync_copy(
            idx_hbm.at[:, pl.ds(wid * cols_w, cols_w)], idx_v)

        def idx_slice(j):
            return idx_v.at[j // k_per_row,
                            pl.ds((j % k_per_row) * CHUNK, CHUNK)]

        for b in range(NBUF):
            pltpu.async_copy(rowsum_hbm.at[idx_slice(b)], bufs[b], sems[b])

        zero = jnp.zeros((LANES,), jnp.float32)

        def group(g, acc):
            for b in range(NBUF):
                j = g * NBUF + b
                pltpu.make_async_copy(
                    rowsum_hbm.at[idx_slice(j)], bufs[b], sems[b]).wait()
                c = bufs[b][0:LANES]
                for r in range(1, CHUNK // LANES):
                    c = c + bufs[b][r * LANES:(r + 1) * LANES]
                acc = acc + c
                nj = j + NBUF

                @pl.when(nj < n_chunks)
                def _():
                    pltpu.async_copy(
                        rowsum_hbm.at[idx_slice(nj)], bufs[b], sems[b])
            return acc

        acc = lax.fori_loop(0, n_chunks // NBUF, group, zero)
        outv[...] = acc
        pltpu.sync_copy(outv, out_hbm.at[wid])

    return sc_kernel


def kernel(x, table):
    nw = 32
    rowsum = _rowsum_tc(table.T)
    idx_t = x.astype(jnp.int32).T
    partials = _make_sc_kernel(nw)(idx_t, rowsum)
    return jnp.sum(partials) / jnp.float32(N_IDX * EMBED)


# speedup vs baseline: 13.4974x; 1.1027x over previous
import functools

import jax
import jax.numpy as jnp
from jax import lax
from jax.experimental import pallas as pl
from jax.experimental.pallas import tpu as pltpu
from jax.experimental.pallas import tpu_sc as plsc

BATCH = 16384
FIELDS = 26
EMBED = 32
VOCAB = 1000000
N_IDX = BATCH * FIELDS
LANES = 16

CHUNK = 128
NBUF = 8
TC_BN = 32768


def _rowsum_tc(table_t):
    def body(t_ref, o_ref):
        o_ref[...] = jnp.sum(t_ref[...], axis=0)

    return pl.pallas_call(
        body,
        grid=(pl.cdiv(VOCAB, TC_BN),),
        in_specs=[pl.BlockSpec((EMBED, TC_BN), lambda i: (0, i))],
        out_specs=pl.BlockSpec((TC_BN,), lambda i: (i,)),
        out_shape=jax.ShapeDtypeStruct((VOCAB,), jnp.float32),
    )(table_t)


def _make_sc_kernel(nw):
    cols_w = BATCH // nw
    n_chunks = FIELDS * (cols_w // CHUNK)
    k_per_row = cols_w // CHUNK
    mesh = plsc.VectorSubcoreMesh(core_axis_name="c", subcore_axis_name="s")
    nc = mesh.num_cores

    @functools.partial(
        pl.kernel,
        out_type=jax.ShapeDtypeStruct((nw, LANES), jnp.float32),
        mesh=mesh,
        compiler_params=pltpu.CompilerParams(use_tc_tiling_on_sc=True),
        scratch_types=[
            pltpu.VMEM((FIELDS, cols_w), jnp.int32),
            pltpu.VMEM((CHUNK,), jnp.float32),
            pltpu.VMEM((CHUNK,), jnp.float32),
            pltpu.VMEM((CHUNK,), jnp.float32),
            pltpu.VMEM((CHUNK,), jnp.float32),
            pltpu.VMEM((CHUNK,), jnp.float32),
            pltpu.VMEM((CHUNK,), jnp.float32),
            pltpu.VMEM((CHUNK,), jnp.float32),
            pltpu.VMEM((CHUNK,), jnp.float32),
            pltpu.VMEM((LANES,), jnp.float32),
            pltpu.SemaphoreType.DMA,
            pltpu.SemaphoreType.DMA,
            pltpu.SemaphoreType.DMA,
            pltpu.SemaphoreType.DMA,
            pltpu.SemaphoreType.DMA,
            pltpu.SemaphoreType.DMA,
            pltpu.SemaphoreType.DMA,
            pltpu.SemaphoreType.DMA,
        ],
    )
    def sc_kernel(idx_hbm, rowsum_hbm, out_hbm,
                  idx_v, buf0, buf1, buf2, buf3, buf4, buf5, buf6, buf7,
                  outv, sem0, sem1, sem2, sem3, sem4, sem5, sem6, sem7):
        bufs = (buf0, buf1, buf2, buf3, buf4, buf5, buf6, buf7)
        sems = (sem0, sem1, sem2, sem3, sem4, sem5, sem6, sem7)
        wid = lax.axis_index("s") * nc + lax.axis_index("c")

        pltpu.sync_copy(
            idx_hbm.at[:, pl.ds(wid * cols_w, cols_w)], idx_v)

        def idx_slice(j):
            return idx_v.at[j // k_per_row,
                            pl.ds((j % k_per_row) * CHUNK, CHUNK)]

        for b in range(NBUF):
            pltpu.async_copy(rowsum_hbm.at[idx_slice(b)], bufs[b], sems[b])

        zero = jnp.zeros((LANES,), jnp.float32)

        def group(g, acc):
            for b in range(NBUF):
                j = g * NBUF + b
                pltpu.make_async_copy(
                    rowsum_hbm.at[idx_slice(j)], bufs[b], sems[b]).wait()
                c = bufs[b][0:LANES]
                for r in range(1, CHUNK // LANES):
                    c = c + bufs[b][r * LANES:(r + 1) * LANES]
                acc = acc + c
                nj = j + NBUF

                @pl.when(nj < n_chunks)
                def _():
                    pltpu.async_copy(
                        rowsum_hbm.at[idx_slice(nj)], bufs[b], sems[b])
            return acc

        acc = lax.fori_loop(0, n_chunks // NBUF, group, zero)
        outv[...] = acc
        pltpu.sync_copy(outv, out_hbm.at[wid])

    return sc_kernel


def kernel(x, table):
    nw = 32
    rowsum = _rowsum_tc(table.T)
    idx_t = x.astype(jnp.int32).T
    partials = _make_sc_kernel(nw)(idx_t, rowsum)
    return jnp.sum(partials) / jnp.float32(N_IDX * EMBED)


# speedup vs baseline: 14.1823x; 1.0507x over previous
import functools

import jax
import jax.numpy as jnp
from jax import lax
from jax.experimental import pallas as pl
from jax.experimental.pallas import tpu as pltpu
from jax.experimental.pallas import tpu_sc as plsc

BATCH = 16384
FIELDS = 26
EMBED = 32
VOCAB = 1000000
N_IDX = BATCH * FIELDS
LANES = 16

CHUNK = 128
NBUF = 8
TC_BN = 65536


def _rowsum_tc(table_t):
    def body(t_ref, o_ref):
        o_ref[...] = jnp.sum(t_ref[...], axis=0)

    return pl.pallas_call(
        body,
        grid=(pl.cdiv(VOCAB, TC_BN),),
        in_specs=[pl.BlockSpec((EMBED, TC_BN), lambda i: (0, i))],
        out_specs=pl.BlockSpec((TC_BN,), lambda i: (i,)),
        out_shape=jax.ShapeDtypeStruct((VOCAB,), jnp.float32),
    )(table_t)


def _make_sc_kernel(nw):
    cols_w = BATCH // nw
    n_chunks = FIELDS * (cols_w // CHUNK)
    k_per_row = cols_w // CHUNK
    mesh = plsc.VectorSubcoreMesh(core_axis_name="c", subcore_axis_name="s")
    nc = mesh.num_cores

    @functools.partial(
        pl.kernel,
        out_type=jax.ShapeDtypeStruct((nw, LANES), jnp.float32),
        mesh=mesh,
        compiler_params=pltpu.CompilerParams(use_tc_tiling_on_sc=True),
        scratch_types=[
            pltpu.VMEM((FIELDS, cols_w), jnp.int32),
            pltpu.VMEM((CHUNK,), jnp.float32),
            pltpu.VMEM((CHUNK,), jnp.float32),
            pltpu.VMEM((CHUNK,), jnp.float32),
            pltpu.VMEM((CHUNK,), jnp.float32),
            pltpu.VMEM((CHUNK,), jnp.float32),
            pltpu.VMEM((CHUNK,), jnp.float32),
            pltpu.VMEM((CHUNK,), jnp.float32),
            pltpu.VMEM((CHUNK,), jnp.float32),
            pltpu.VMEM((LANES,), jnp.float32),
            pltpu.SemaphoreType.DMA,
            pltpu.SemaphoreType.DMA,
            pltpu.SemaphoreType.DMA,
            pltpu.SemaphoreType.DMA,
            pltpu.SemaphoreType.DMA,
            pltpu.SemaphoreType.DMA,
            pltpu.SemaphoreType.DMA,
            pltpu.SemaphoreType.DMA,
        ],
    )
    def sc_kernel(idx_hbm, rowsum_hbm, out_hbm,
                  idx_v, buf0, buf1, buf2, buf3, buf4, buf5, buf6, buf7,
                  outv, sem0, sem1, sem2, sem3, sem4, sem5, sem6, sem7):
        bufs = (buf0, buf1, buf2, buf3, buf4, buf5, buf6, buf7)
        sems = (sem0, sem1, sem2, sem3, sem4, sem5, sem6, sem7)
        wid = lax.axis_index("s") * nc + lax.axis_index("c")

        pltpu.sync_copy(
            idx_hbm.at[:, pl.ds(wid * cols_w, cols_w)], idx_v)

        def idx_slice(j):
            return idx_v.at[j // k_per_row,
                            pl.ds((j % k_per_row) * CHUNK, CHUNK)]

        for b in range(NBUF):
            pltpu.async_copy(rowsum_hbm.at[idx_slice(b)], bufs[b], sems[b])

        zero = jnp.zeros((LANES,), jnp.float32)

        def group(g, acc):
            for b in range(NBUF):
                j = g * NBUF + b
                pltpu.make_async_copy(
                    rowsum_hbm.at[idx_slice(j)], bufs[b], sems[b]).wait()
                c = bufs[b][0:LANES]
                for r in range(1, CHUNK // LANES):
                    c = c + bufs[b][r * LANES:(r + 1) * LANES]
                acc = acc + c
                nj = j + NBUF

                @pl.when(nj < n_chunks)
                def _():
                    pltpu.async_copy(
                        rowsum_hbm.at[idx_slice(nj)], bufs[b], sems[b])
            return acc

        acc = lax.fori_loop(0, n_chunks // NBUF, group, zero)
        outv[...] = acc
        pltpu.sync_copy(outv, out_hbm.at[wid])

    return sc_kernel


def kernel(x, table):
    nw = 32
    rowsum = _rowsum_tc(table.T)
    idx_t = x.astype(jnp.int32).T
    partials = _make_sc_kernel(nw)(idx_t, rowsum)
    return jnp.sum(partials) / jnp.float32(N_IDX * EMBED)


# speedup vs baseline: 14.1949x; 1.0009x over previous
import functools

import jax
import jax.numpy as jnp
from jax import lax
from jax.experimental import pallas as pl
from jax.experimental.pallas import tpu as pltpu
from jax.experimental.pallas import tpu_sc as plsc

BATCH = 16384
FIELDS = 26
EMBED = 32
VOCAB = 1000000
N_IDX = BATCH * FIELDS
LANES = 16

CHUNK = 128
NBUF = 8
TC_BN = 49152


def _rowsum_tc(table_t):
    def body(t_ref, o_ref):
        o_ref[...] = jnp.sum(t_ref[...], axis=0)

    return pl.pallas_call(
        body,
        grid=(pl.cdiv(VOCAB, TC_BN),),
        in_specs=[pl.BlockSpec((EMBED, TC_BN), lambda i: (0, i))],
        out_specs=pl.BlockSpec((TC_BN,), lambda i: (i,)),
        out_shape=jax.ShapeDtypeStruct((VOCAB,), jnp.float32),
    )(table_t)


def _make_sc_kernel(nw):
    cols_w = BATCH // nw
    n_chunks = FIELDS * (cols_w // CHUNK)
    k_per_row = cols_w // CHUNK
    mesh = plsc.VectorSubcoreMesh(core_axis_name="c", subcore_axis_name="s")
    nc = mesh.num_cores

    @functools.partial(
        pl.kernel,
        out_type=jax.ShapeDtypeStruct((nw, LANES), jnp.float32),
        mesh=mesh,
        compiler_params=pltpu.CompilerParams(use_tc_tiling_on_sc=True),
        scratch_types=[
            pltpu.VMEM((FIELDS, cols_w), jnp.int32),
            pltpu.VMEM((CHUNK,), jnp.float32),
            pltpu.VMEM((CHUNK,), jnp.float32),
            pltpu.VMEM((CHUNK,), jnp.float32),
            pltpu.VMEM((CHUNK,), jnp.float32),
            pltpu.VMEM((CHUNK,), jnp.float32),
            pltpu.VMEM((CHUNK,), jnp.float32),
            pltpu.VMEM((CHUNK,), jnp.float32),
            pltpu.VMEM((CHUNK,), jnp.float32),
            pltpu.VMEM((LANES,), jnp.float32),
            pltpu.SemaphoreType.DMA,
            pltpu.SemaphoreType.DMA,
            pltpu.SemaphoreType.DMA,
            pltpu.SemaphoreType.DMA,
            pltpu.SemaphoreType.DMA,
            pltpu.SemaphoreType.DMA,
            pltpu.SemaphoreType.DMA,
            pltpu.SemaphoreType.DMA,
        ],
    )
    def sc_kernel(idx_hbm, rowsum_hbm, out_hbm,
                  idx_v, buf0, buf1, buf2, buf3, buf4, buf5, buf6, buf7,
                  outv, sem0, sem1, sem2, sem3, sem4, sem5, sem6, sem7):
        bufs = (buf0, buf1, buf2, buf3, buf4, buf5, buf6, buf7)
        sems = (sem0, sem1, sem2, sem3, sem4, sem5, sem6, sem7)
        wid = lax.axis_index("s") * nc + lax.axis_index("c")

        pltpu.sync_copy(
            idx_hbm.at[:, pl.ds(wid * cols_w, cols_w)], idx_v)

        def idx_slice(j):
            return idx_v.at[j // k_per_row,
                            pl.ds((j % k_per_row) * CHUNK, CHUNK)]

        for b in range(NBUF):
            pltpu.async_copy(rowsum_hbm.at[idx_slice(b)], bufs[b], sems[b])

        zero = jnp.zeros((LANES,), jnp.float32)

        def group(g, acc):
            for b in range(NBUF):
                j = g * NBUF + b
                pltpu.make_async_copy(
                    rowsum_hbm.at[idx_slice(j)], bufs[b], sems[b]).wait()
                c = bufs[b][0:LANES]
                for r in range(1, CHUNK // LANES):
                    c = c + bufs[b][r * LANES:(r + 1) * LANES]
                acc = acc + c
                nj = j + NBUF

                @pl.when(nj < n_chunks)
                def _():
                    pltpu.async_copy(
                        rowsum_hbm.at[idx_slice(nj)], bufs[b], sems[b])
            return acc

        acc = lax.fori_loop(0, n_chunks // NBUF, group, zero)
        outv[...] = acc
        pltpu.sync_copy(outv, out_hbm.at[wid])

    return sc_kernel


def kernel(x, table):
    nw = 32
    rowsum = _rowsum_tc(table.T)
    idx_t = x.astype(jnp.int32).T
    partials = _make_sc_kernel(nw)(idx_t, rowsum)
    return jnp.sum(partials) / jnp.float32(N_IDX * EMBED)
